# baseline (speedup 1.0000x reference)
import functools

import jax
import jax.numpy as jnp
from jax import lax
from jax.experimental import pallas as pl
from jax.experimental.pallas import tpu as pltpu
from jax.experimental.pallas import tpu_sc as plsc

N = 4096
K = 20
BLK = 512
GRID = N // BLK
OC = 64

NUM_WORKERS = 32
PTS_PER_WORKER = N // NUM_WORKERS


R = 7
NSLAB = 32
SLAB = N // NSLAB


def _knn_body(x_ref, Wc_ref, Wb_ref, b_ref, idxT_ref, a_ref, bf_ref,
              dist_ref, d2_ref, cv_ref, ci_ref):
    i = pl.program_id(0)
    xb = x_ref[pl.ds(i * BLK, BLK), :]
    a_ref[...] = jnp.dot(xb, Wc_ref[...],
                         preferred_element_type=jnp.float32) + b_ref[...]
    bf_ref[...] = jnp.dot(xb, Wb_ref[...],
                          preferred_element_type=jnp.float32)

    xf = x_ref[...]
    g = lax.dot_general(xf, xb, (((1,), (1,)), ((), ())),
                        preferred_element_type=jnp.float32)
    sq_col = jnp.sum(xf * xf, axis=1, keepdims=True)
    sq_row = lax.dot_general(jnp.ones((1, xb.shape[1]), jnp.float32),
                             xb * xb, (((1,), (1,)), ((), ())),
                             preferred_element_type=jnp.float32)
    dist = sq_col - 2.0 * g + sq_row
    rows = lax.broadcasted_iota(jnp.int32, (N, BLK), 0)
    cols = lax.broadcasted_iota(jnp.int32, (1, BLK), 1) + i * BLK
    dist = jnp.where(rows == cols, dist + 1e10, dist)
    dist_ref[...] = dist

    big = jnp.float32(jnp.inf)

    rows128 = lax.broadcasted_iota(jnp.int32, (SLAB, BLK), 0)
    for r in range(R):
        for s in range(NSLAB):
            src = dist_ref if r == 0 else d2_ref
            sl = src[pl.ds(s * SLAB, SLAB), :]
            mval = jnp.min(sl, axis=0, keepdims=True)
            aidx = jnp.min(jnp.where(sl == mval, rows128, jnp.int32(SLAB)),
                           axis=0, keepdims=True)
            if r + 1 < R:
                d2_ref[pl.ds(s * SLAB, SLAB), :] = jnp.where(
                    rows128 == aidx, big, sl)
            cv_ref[r, pl.ds(s, 1), :] = mval
            ci_ref[r, pl.ds(s, 1), :] = aidx + s * SLAB

    iota32 = lax.broadcasted_iota(jnp.int32, (NSLAB, BLK), 0)
    H = cv_ref[0]
    Hi = ci_ref[0]
    lvl = jnp.zeros((NSLAB, BLK), jnp.int32)
    for t in range(K):
        m = jnp.min(H, axis=0, keepdims=True)
        ji = jnp.min(jnp.where(H == m, Hi, jnp.int32(N)),
                     axis=0, keepdims=True)
        idxT_ref[pl.ds(t, 1), :] = ji
        is_ws = (H == m) & (Hi == ji)
        lvl = lvl + is_ws.astype(jnp.int32)
        nxtv = jnp.full((NSLAB, BLK), big, jnp.float32)
        nxti = jnp.full((NSLAB, BLK), N, jnp.int32)
        for r in range(1, R):
            sel_r = lvl == r
            nxtv = jnp.where(sel_r, cv_ref[r], nxtv)
            nxti = jnp.where(sel_r, ci_ref[r], nxti)
        H = jnp.where(is_ws, nxtv, H)
        Hi = jnp.where(is_ws, nxti, Hi)

    exhausted = jnp.any(lvl >= R)

    @pl.when(exhausted)
    def _fallback():
        for t in range(K):
            d = dist_ref[...]
            m = jnp.min(d, axis=0, keepdims=True)
            sel = jnp.where(d == m, rows, jnp.int32(N))
            j = jnp.min(sel, axis=0, keepdims=True)
            idxT_ref[pl.ds(t, 1), :] = j
            dist_ref[...] = jnp.where(rows == j, big, d)


def _knn_call(C):
    return pl.pallas_call(
        _knn_body,
        grid=(GRID,),
        in_specs=[
            pl.BlockSpec((N, C), lambda i: (0, 0)),
            pl.BlockSpec((C, OC), lambda i: (0, 0)),
            pl.BlockSpec((C, 128), lambda i: (0, 0)),
            pl.BlockSpec((1, OC), lambda i: (0, 0)),
        ],
        out_specs=[
            pl.BlockSpec((K, BLK), lambda i: (0, i)),
            pl.BlockSpec((BLK, OC), lambda i: (i, 0)),
            pl.BlockSpec((BLK, 128), lambda i: (i, 0)),
        ],
        out_shape=[
            jax.ShapeDtypeStruct((K, N), jnp.int32),
            jax.ShapeDtypeStruct((N, OC), jnp.float32),
            jax.ShapeDtypeStruct((N, 128), jnp.float32),
        ],
        scratch_shapes=[pltpu.VMEM((N, BLK), jnp.float32),
                        pltpu.VMEM((N, BLK), jnp.float32),
                        pltpu.VMEM((R, NSLAB, BLK), jnp.float32),
                        pltpu.VMEM((R, NSLAB, BLK), jnp.int32)],
        compiler_params=pltpu.CompilerParams(
            dimension_semantics=("arbitrary",),
            vmem_limit_bytes=100 * 1024 * 1024),
    )


def _make_gather_max(residual):
    mesh = plsc.VectorSubcoreMesh(core_axis_name="c", subcore_axis_name="s")
    P = PTS_PER_WORKER
    scratch = [
        pltpu.VMEM((K, P), jnp.int32),
        pltpu.VMEM((P, 128), jnp.float32),
        pltpu.VMEM((P, 128), jnp.float32),
        pltpu.VMEM((P, 128), jnp.float32),
        pltpu.VMEM((P, 128), jnp.float32),
        pltpu.VMEM((P, OC), jnp.float32),
        pltpu.VMEM((P, OC), jnp.float32),
    ]
    if residual:
        scratch.append(pltpu.VMEM((P, OC), jnp.float32))
    scratch += [pltpu.SemaphoreType.DMA] * 4
    UNROLL = 4

    def body(*refs):
        if residual:
            (idxT_hbm, bf_hbm, a_hbm, xp_hbm, out_hbm, idx_v,
             buf0, buf1, buf2, buf3, acc, a_v, xp_v,
             sem0, sem1, sem2, sem3) = refs
        else:
            (idxT_hbm, bf_hbm, a_hbm, out_hbm, idx_v,
             buf0, buf1, buf2, buf3, acc, a_v,
             sem0, sem1, sem2, sem3) = refs
            xp_hbm = xp_v = None
        wid = lax.axis_index("s") * 2 + lax.axis_index("c")
        base = wid * P
        pltpu.sync_copy(idxT_hbm.at[:, pl.ds(base, P)], idx_v)
        pltpu.sync_copy(a_hbm.at[pl.ds(base, P)], a_v)
        if residual:
            pltpu.sync_copy(xp_hbm.at[pl.ds(base, P)], xp_v)

        bufs = (buf0, buf1, buf2, buf3)
        sems = (sem0, sem1, sem2, sem3)
        copies = [None] * 4
        for t in range(4):
            copies[t] = pltpu.async_copy(bf_hbm.at[idx_v.at[t]], bufs[t],
                                         sems[t])
        for t in range(0, K, 2):
            copies[t % 4].wait()
            copies[(t + 1) % 4].wait()
            ba, bb = bufs[t % 4], bufs[(t + 1) % 4]

            def pair_body(q, _, ba=ba, bb=bb, first=(t == 0)):
                for u in range(UNROLL):
                    p = q * UNROLL + u
                    for gch in range(OC // 16):
                        sl = pl.ds(gch * 16, 16)
                        v = jnp.maximum(ba[p, sl], bb[p, sl])
                        if not first:
                            v = jnp.maximum(acc[p, sl], v)
                        acc[p, sl] = v
                return 0
            lax.fori_loop(0, P // UNROLL, pair_body, 0)
            for tn in (t + 4, t + 5):
                if tn < K:
                    copies[tn % 4] = pltpu.async_copy(
                        bf_hbm.at[idx_v.at[tn]], bufs[tn % 4], sems[tn % 4])

        def epi_body(p, _):
            for gch in range(OC // 16):
                sl = pl.ds(gch * 16, 16)
                v = jnp.maximum(a_v[p, sl] + acc[p, sl], jnp.float32(0.0))
                if residual:
                    v = v - xp_v[p, sl]
                acc[p, sl] = v
            return 0
        lax.fori_loop(0, P, epi_body, 0)
        pltpu.sync_copy(acc, out_hbm.at[pl.ds(base, P)])

    return functools.partial(
        pl.kernel, body, mesh=mesh,
        out_type=jax.ShapeDtypeStruct((N, OC), jnp.float32),
        scratch_types=scratch)


def _head_body(x1_ref, x2_ref, x3_ref, A_ref, B_ref, C_ref, bl1_ref,
               Wm1_ref, bm1_ref, Wm2_ref, bm2_ref, Wm3_ref, bm3_ref,
               out_ref):
    dot = functools.partial(jnp.dot, preferred_element_type=jnp.float32)
    h = (dot(x1_ref[...], A_ref[...]) + dot(x2_ref[...], B_ref[...]) +
         dot(x3_ref[...], C_ref[...]) + bl1_ref[...])
    h = jnp.maximum(h, 0.0)
    h = jnp.maximum(dot(h, Wm1_ref[...]) + bm1_ref[...], 0.0)
    h = jnp.maximum(dot(h, Wm2_ref[...]) + bm2_ref[...], 0.0)
    logits = dot(h, Wm3_ref[...]) + bm3_ref[...]
    m = jnp.max(logits, axis=1, keepdims=True)
    s = jnp.sum(jnp.exp(logits - m), axis=1, keepdims=True)
    out_ref[...] = logits - (m + jnp.log(s))


def _head_call():
    full = lambda shape: pl.BlockSpec(shape, lambda i: (0, 0))
    return pl.pallas_call(
        _head_body,
        grid=(GRID,),
        in_specs=[
            pl.BlockSpec((BLK, OC), lambda i: (i, 0)),
            pl.BlockSpec((BLK, OC), lambda i: (i, 0)),
            pl.BlockSpec((BLK, OC), lambda i: (i, 0)),
            full((OC, 1024)), full((OC, 1024)), full((OC, 1024)),
            full((1, 1024)),
            full((1024, 256)), full((1, 256)),
            full((256, 128)), full((1, 128)),
            full((128, 64)), full((1, 64)),
        ],
        out_specs=pl.BlockSpec((BLK, 64), lambda i: (i, 0)),
        out_shape=jax.ShapeDtypeStruct((N, 64), jnp.float32),
        compiler_params=pltpu.CompilerParams(
            dimension_semantics=("arbitrary",)),
    )


def _split_edge_weights(W, C, pad_to):
    Wa, Wb = W[:C], W[C:]
    Wc = Wa - Wb
    if pad_to > C:
        padw = ((0, pad_to - C), (0, 0))
        Wc = jnp.pad(Wc, padw)
        Wb = jnp.pad(Wb, padw)
    Wb = jnp.pad(Wb, ((0, 0), (0, 128 - Wb.shape[1])))
    return Wc, Wb


def kernel(x, W1, b1, Wd1, bd1, Wd2, bd2, Wd3, bd3, Wl1, bl1, Wm1, bm1,
           Wm2, bm2, Wm3, bm3):
    gmax_plain = _make_gather_max(False)()
    gmax_res = _make_gather_max(True)()

    x0 = jnp.pad(x, ((0, 0), (0, 5)))
    Wc0, Wb0 = _split_edge_weights(W1, 3, 8)
    idxT0, a0, bf0 = _knn_call(8)(x0, Wc0, Wb0, b1[None, :])
    f = gmax_plain(idxT0, bf0, a0)

    feats = []
    cur = f
    for Wd, bd in ((Wd1, bd1), (Wd2, bd2), (Wd3, bd3)):
        Wc, Wb = _split_edge_weights(Wd, OC, OC)
        idxT, a, bf = _knn_call(OC)(cur, Wc, Wb, bd[None, :])
        nxt = gmax_res(idxT, bf, a, cur)
        feats.append(nxt)
        cur = nxt

    x1, x2, x3 = feats
    A, B, C = Wl1[:OC], Wl1[OC:2 * OC], Wl1[2 * OC:]
    Wm3p = jnp.pad(Wm3, ((0, 0), (0, 64 - Wm3.shape[1])))
    bm3p = jnp.pad(bm3, (0, 64 - bm3.shape[0]),
                   constant_values=-1e30)
    out = _head_call()(x1, x2, x3, A, B, C, bl1[None, :],
                       Wm1, bm1[None, :], Wm2, bm2[None, :],
                       Wm3p, bm3p[None, :])
    return out[:, :bm3.shape[0]]


# speedup vs baseline: 1.1031x; 1.1031x over previous
import functools

import jax
import jax.numpy as jnp
from jax import lax
from jax.experimental import pallas as pl
from jax.experimental.pallas import tpu as pltpu
from jax.experimental.pallas import tpu_sc as plsc

N = 4096
K = 20
BLK = 512
GRID = N // BLK
OC = 64

NUM_WORKERS = 32
PTS_PER_WORKER = N // NUM_WORKERS


R = 7
NSLAB = 32
SLAB = N // NSLAB


def _knn_body(x_ref, xT_ref, Wc_ref, Wb_ref, b_ref, idxT_ref, a_ref, bf_ref,
              dist_ref, d2_ref, cv_ref, ci_ref):
    i = pl.program_id(0)
    xb = x_ref[pl.ds(i * BLK, BLK), :]
    a_ref[...] = jnp.dot(xb, Wc_ref[...],
                         preferred_element_type=jnp.float32) + b_ref[...]
    bf_ref[...] = jnp.dot(xb, Wb_ref[...],
                          preferred_element_type=jnp.float32)

    xf = x_ref[...]
    xT = xT_ref[...]
    g = jnp.dot(xf, xT, preferred_element_type=jnp.float32)
    sq_col = jnp.sum(xf * xf, axis=1, keepdims=True)
    sq_row = jnp.sum(xT * xT, axis=0, keepdims=True)
    dist = sq_col - 2.0 * g + sq_row
    rows = lax.broadcasted_iota(jnp.int32, (N, BLK), 0)
    cols = lax.broadcasted_iota(jnp.int32, (1, BLK), 1) + i * BLK
    dist = jnp.where(rows == cols, dist + 1e10, dist)
    dist_ref[...] = dist

    big = jnp.float32(jnp.inf)

    rows128 = lax.broadcasted_iota(jnp.int32, (SLAB, BLK), 0)
    for r in range(R):
        for s in range(NSLAB):
            src = dist_ref if r == 0 else d2_ref
            sl = src[pl.ds(s * SLAB, SLAB), :]
            mval = jnp.min(sl, axis=0, keepdims=True)
            aidx = jnp.min(jnp.where(sl == mval, rows128, jnp.int32(SLAB)),
                           axis=0, keepdims=True)
            if r + 1 < R:
                d2_ref[pl.ds(s * SLAB, SLAB), :] = jnp.where(
                    rows128 == aidx, big, sl)
            cv_ref[r, pl.ds(s, 1), :] = mval
            ci_ref[r, pl.ds(s, 1), :] = aidx + s * SLAB

    iota32 = lax.broadcasted_iota(jnp.int32, (NSLAB, BLK), 0)
    H = cv_ref[0]
    Hi = ci_ref[0]
    lvl = jnp.zeros((NSLAB, BLK), jnp.int32)
    for t in range(K):
        m = jnp.min(H, axis=0, keepdims=True)
        ji = jnp.min(jnp.where(H == m, Hi, jnp.int32(N)),
                     axis=0, keepdims=True)
        idxT_ref[pl.ds(t, 1), :] = ji
        is_ws = (H == m) & (Hi == ji)
        lvl = lvl + is_ws.astype(jnp.int32)
        nxtv = jnp.full((NSLAB, BLK), big, jnp.float32)
        nxti = jnp.full((NSLAB, BLK), N, jnp.int32)
        for r in range(1, R):
            sel_r = lvl == r
            nxtv = jnp.where(sel_r, cv_ref[r], nxtv)
            nxti = jnp.where(sel_r, ci_ref[r], nxti)
        H = jnp.where(is_ws, nxtv, H)
        Hi = jnp.where(is_ws, nxti, Hi)

    exhausted = jnp.any(lvl >= R)

    @pl.when(exhausted)
    def _fallback():
        for t in range(K):
            d = dist_ref[...]
            m = jnp.min(d, axis=0, keepdims=True)
            sel = jnp.where(d == m, rows, jnp.int32(N))
            j = jnp.min(sel, axis=0, keepdims=True)
            idxT_ref[pl.ds(t, 1), :] = j
            dist_ref[...] = jnp.where(rows == j, big, d)


def _knn_call(C):
    return pl.pallas_call(
        _knn_body,
        grid=(GRID,),
        in_specs=[
            pl.BlockSpec((N, C), lambda i: (0, 0)),
            pl.BlockSpec((C, BLK), lambda i: (0, i)),
            pl.BlockSpec((C, OC), lambda i: (0, 0)),
            pl.BlockSpec((C, 128), lambda i: (0, 0)),
            pl.BlockSpec((1, OC), lambda i: (0, 0)),
        ],
        out_specs=[
            pl.BlockSpec((K, BLK), lambda i: (0, i)),
            pl.BlockSpec((BLK, OC), lambda i: (i, 0)),
            pl.BlockSpec((BLK, 128), lambda i: (i, 0)),
        ],
        out_shape=[
            jax.ShapeDtypeStruct((K, N), jnp.int32),
            jax.ShapeDtypeStruct((N, OC), jnp.float32),
            jax.ShapeDtypeStruct((N, 128), jnp.float32),
        ],
        scratch_shapes=[pltpu.VMEM((N, BLK), jnp.float32),
                        pltpu.VMEM((N, BLK), jnp.float32),
                        pltpu.VMEM((R, NSLAB, BLK), jnp.float32),
                        pltpu.VMEM((R, NSLAB, BLK), jnp.int32)],
        compiler_params=pltpu.CompilerParams(
            dimension_semantics=("arbitrary",),
            vmem_limit_bytes=100 * 1024 * 1024),
    )


def _make_gather_max(residual):
    mesh = plsc.VectorSubcoreMesh(core_axis_name="c", subcore_axis_name="s")
    P = PTS_PER_WORKER
    scratch = [
        pltpu.VMEM((K, P), jnp.int32),
        pltpu.VMEM((P, 128), jnp.float32),
        pltpu.VMEM((P, 128), jnp.float32),
        pltpu.VMEM((P, 128), jnp.float32),
        pltpu.VMEM((P, 128), jnp.float32),
        pltpu.VMEM((P, OC), jnp.float32),
        pltpu.VMEM((P, OC), jnp.float32),
    ]
    if residual:
        scratch.append(pltpu.VMEM((P, OC), jnp.float32))
    scratch += [pltpu.SemaphoreType.DMA] * 4
    UNROLL = 4

    def body(*refs):
        if residual:
            (idxT_hbm, bf_hbm, a_hbm, xp_hbm, out_hbm, idx_v,
             buf0, buf1, buf2, buf3, acc, a_v, xp_v,
             sem0, sem1, sem2, sem3) = refs
        else:
            (idxT_hbm, bf_hbm, a_hbm, out_hbm, idx_v,
             buf0, buf1, buf2, buf3, acc, a_v,
             sem0, sem1, sem2, sem3) = refs
            xp_hbm = xp_v = None
        wid = lax.axis_index("s") * 2 + lax.axis_index("c")
        base = wid * P
        pltpu.sync_copy(idxT_hbm.at[:, pl.ds(base, P)], idx_v)
        pltpu.sync_copy(a_hbm.at[pl.ds(base, P)], a_v)
        if residual:
            pltpu.sync_copy(xp_hbm.at[pl.ds(base, P)], xp_v)

        bufs = (buf0, buf1, buf2, buf3)
        sems = (sem0, sem1, sem2, sem3)
        copies = [None] * 4
        for t in range(4):
            copies[t] = pltpu.async_copy(bf_hbm.at[idx_v.at[t]], bufs[t],
                                         sems[t])
        for t in range(0, K, 2):
            copies[t % 4].wait()
            copies[(t + 1) % 4].wait()
            ba, bb = bufs[t % 4], bufs[(t + 1) % 4]

            def pair_body(q, _, ba=ba, bb=bb, first=(t == 0)):
                for u in range(UNROLL):
                    p = q * UNROLL + u
                    for gch in range(OC // 16):
                        sl = pl.ds(gch * 16, 16)
                        v = jnp.maximum(ba[p, sl], bb[p, sl])
                        if not first:
                            v = jnp.maximum(acc[p, sl], v)
                        acc[p, sl] = v
                return 0
            lax.fori_loop(0, P // UNROLL, pair_body, 0)
            for tn in (t + 4, t + 5):
                if tn < K:
                    copies[tn % 4] = pltpu.async_copy(
                        bf_hbm.at[idx_v.at[tn]], bufs[tn % 4], sems[tn % 4])

        def epi_body(p, _):
            for gch in range(OC // 16):
                sl = pl.ds(gch * 16, 16)
                v = jnp.maximum(a_v[p, sl] + acc[p, sl], jnp.float32(0.0))
                if residual:
                    v = v - xp_v[p, sl]
                acc[p, sl] = v
            return 0
        lax.fori_loop(0, P, epi_body, 0)
        pltpu.sync_copy(acc, out_hbm.at[pl.ds(base, P)])

    return functools.partial(
        pl.kernel, body, mesh=mesh,
        out_type=jax.ShapeDtypeStruct((N, OC), jnp.float32),
        scratch_types=scratch)


def _head_body(x1_ref, x2_ref, x3_ref, A_ref, B_ref, C_ref, bl1_ref,
               Wm1_ref, bm1_ref, Wm2_ref, bm2_ref, Wm3_ref, bm3_ref,
               out_ref):
    dot = functools.partial(jnp.dot, preferred_element_type=jnp.float32)
    h = (dot(x1_ref[...], A_ref[...]) + dot(x2_ref[...], B_ref[...]) +
         dot(x3_ref[...], C_ref[...]) + bl1_ref[...])
    h = jnp.maximum(h, 0.0)
    h = jnp.maximum(dot(h, Wm1_ref[...]) + bm1_ref[...], 0.0)
    h = jnp.maximum(dot(h, Wm2_ref[...]) + bm2_ref[...], 0.0)
    logits = dot(h, Wm3_ref[...]) + bm3_ref[...]
    m = jnp.max(logits, axis=1, keepdims=True)
    s = jnp.sum(jnp.exp(logits - m), axis=1, keepdims=True)
    out_ref[...] = logits - (m + jnp.log(s))


def _head_call():
    full = lambda shape: pl.BlockSpec(shape, lambda i: (0, 0))
    return pl.pallas_call(
        _head_body,
        grid=(GRID,),
        in_specs=[
            pl.BlockSpec((BLK, OC), lambda i: (i, 0)),
            pl.BlockSpec((BLK, OC), lambda i: (i, 0)),
            pl.BlockSpec((BLK, OC), lambda i: (i, 0)),
            full((OC, 1024)), full((OC, 1024)), full((OC, 1024)),
            full((1, 1024)),
            full((1024, 256)), full((1, 256)),
            full((256, 128)), full((1, 128)),
            full((128, 64)), full((1, 64)),
        ],
        out_specs=pl.BlockSpec((BLK, 64), lambda i: (i, 0)),
        out_shape=jax.ShapeDtypeStruct((N, 64), jnp.float32),
        compiler_params=pltpu.CompilerParams(
            dimension_semantics=("arbitrary",)),
    )


def _split_edge_weights(W, C, pad_to):
    Wa, Wb = W[:C], W[C:]
    Wc = Wa - Wb
    if pad_to > C:
        padw = ((0, pad_to - C), (0, 0))
        Wc = jnp.pad(Wc, padw)
        Wb = jnp.pad(Wb, padw)
    Wb = jnp.pad(Wb, ((0, 0), (0, 128 - Wb.shape[1])))
    return Wc, Wb


def kernel(x, W1, b1, Wd1, bd1, Wd2, bd2, Wd3, bd3, Wl1, bl1, Wm1, bm1,
           Wm2, bm2, Wm3, bm3):
    gmax_plain = _make_gather_max(False)()
    gmax_res = _make_gather_max(True)()

    x0 = jnp.pad(x, ((0, 0), (0, 5)))
    Wc0, Wb0 = _split_edge_weights(W1, 3, 8)
    idxT0, a0, bf0 = _knn_call(8)(x0, x0.T, Wc0, Wb0, b1[None, :])
    f = gmax_plain(idxT0, bf0, a0)

    feats = []
    cur = f
    for Wd, bd in ((Wd1, bd1), (Wd2, bd2), (Wd3, bd3)):
        Wc, Wb = _split_edge_weights(Wd, OC, OC)
        idxT, a, bf = _knn_call(OC)(cur, cur.T, Wc, Wb, bd[None, :])
        nxt = gmax_res(idxT, bf, a, cur)
        feats.append(nxt)
        cur = nxt

    x1, x2, x3 = feats
    A, B, C = Wl1[:OC], Wl1[OC:2 * OC], Wl1[2 * OC:]
    Wm3p = jnp.pad(Wm3, ((0, 0), (0, 64 - Wm3.shape[1])))
    bm3p = jnp.pad(bm3, (0, 64 - bm3.shape[0]),
                   constant_values=-1e30)
    out = _head_call()(x1, x2, x3, A, B, C, bl1[None, :],
                       Wm1, bm1[None, :], Wm2, bm2[None, :],
                       Wm3p, bm3p[None, :])
    return out[:, :bm3.shape[0]]


# speedup vs baseline: 1.1384x; 1.0321x over previous
import functools

import jax
import jax.numpy as jnp
from jax import lax
from jax.experimental import pallas as pl
from jax.experimental.pallas import tpu as pltpu
from jax.experimental.pallas import tpu_sc as plsc

N = 4096
K = 20
BLK = 512
GRID = N // BLK
OC = 64

NUM_WORKERS = 32
PTS_PER_WORKER = N // NUM_WORKERS


R = 7
NSLAB = 32
SLAB = N // NSLAB


def _knn_body(x_ref, xT_ref, Wc_ref, Wb_ref, b_ref, idxT_ref, a_ref, bf_ref,
              dist_ref, d2_ref, cv_ref, ci_ref):
    i = pl.program_id(0)
    xb = x_ref[pl.ds(i * BLK, BLK), :]
    a_ref[...] = jnp.dot(xb, Wc_ref[...],
                         preferred_element_type=jnp.float32) + b_ref[...]
    bf_ref[...] = jnp.dot(xb, Wb_ref[...],
                          preferred_element_type=jnp.float32)

    xf = x_ref[...]
    xT = xT_ref[...]
    g = jnp.dot(xf, xT, preferred_element_type=jnp.float32)
    sq_col = jnp.sum(xf * xf, axis=1, keepdims=True)
    sq_row = jnp.sum(xT * xT, axis=0, keepdims=True)
    dist_ref[...] = sq_col - 2.0 * g + sq_row
    db = dist_ref[pl.ds(i * BLK, BLK), :]
    diag = (lax.broadcasted_iota(jnp.int32, (BLK, BLK), 0) ==
            lax.broadcasted_iota(jnp.int32, (BLK, BLK), 1))
    dist_ref[pl.ds(i * BLK, BLK), :] = jnp.where(diag, db + 1e10, db)

    big = jnp.float32(jnp.inf)

    rows128 = lax.broadcasted_iota(jnp.int32, (SLAB, BLK), 0)
    for r in range(R):
        for s in range(NSLAB):
            src = dist_ref if r == 0 else d2_ref
            sl = src[pl.ds(s * SLAB, SLAB), :]
            mval = jnp.min(sl, axis=0, keepdims=True)
            aidx = jnp.min(jnp.where(sl == mval, rows128, jnp.int32(SLAB)),
                           axis=0, keepdims=True)
            if r + 1 < R:
                d2_ref[pl.ds(s * SLAB, SLAB), :] = jnp.where(
                    rows128 == aidx, big, sl)
            cv_ref[r, pl.ds(s, 1), :] = mval
            ci_ref[r, pl.ds(s, 1), :] = aidx + s * SLAB

    iota32 = lax.broadcasted_iota(jnp.int32, (NSLAB, BLK), 0)
    H = cv_ref[0]
    Hi = ci_ref[0]
    lvl = jnp.zeros((NSLAB, BLK), jnp.int32)
    for t in range(K):
        m = jnp.min(H, axis=0, keepdims=True)
        ji = jnp.min(jnp.where(H == m, Hi, jnp.int32(N)),
                     axis=0, keepdims=True)
        idxT_ref[pl.ds(t, 1), :] = ji
        is_ws = (H == m) & (Hi == ji)
        lvl = lvl + is_ws.astype(jnp.int32)
        nxtv = jnp.full((NSLAB, BLK), big, jnp.float32)
        nxti = jnp.full((NSLAB, BLK), N, jnp.int32)
        for r in range(1, R):
            sel_r = lvl == r
            nxtv = jnp.where(sel_r, cv_ref[r], nxtv)
            nxti = jnp.where(sel_r, ci_ref[r], nxti)
        H = jnp.where(is_ws, nxtv, H)
        Hi = jnp.where(is_ws, nxti, Hi)

    exhausted = jnp.any(lvl >= R)

    @pl.when(exhausted)
    def _fallback():
        rows = lax.broadcasted_iota(jnp.int32, (N, BLK), 0)
        for t in range(K):
            d = dist_ref[...]
            m = jnp.min(d, axis=0, keepdims=True)
            sel = jnp.where(d == m, rows, jnp.int32(N))
            j = jnp.min(sel, axis=0, keepdims=True)
            idxT_ref[pl.ds(t, 1), :] = j
            dist_ref[...] = jnp.where(rows == j, big, d)


def _knn_call(C):
    return pl.pallas_call(
        _knn_body,
        grid=(GRID,),
        in_specs=[
            pl.BlockSpec((N, C), lambda i: (0, 0)),
            pl.BlockSpec((C, BLK), lambda i: (0, i)),
            pl.BlockSpec((C, OC), lambda i: (0, 0)),
            pl.BlockSpec((C, 128), lambda i: (0, 0)),
            pl.BlockSpec((1, OC), lambda i: (0, 0)),
        ],
        out_specs=[
            pl.BlockSpec((K, BLK), lambda i: (0, i)),
            pl.BlockSpec((BLK, OC), lambda i: (i, 0)),
            pl.BlockSpec((BLK, 128), lambda i: (i, 0)),
        ],
        out_shape=[
            jax.ShapeDtypeStruct((K, N), jnp.int32),
            jax.ShapeDtypeStruct((N, OC), jnp.float32),
            jax.ShapeDtypeStruct((N, 128), jnp.float32),
        ],
        scratch_shapes=[pltpu.VMEM((N, BLK), jnp.float32),
                        pltpu.VMEM((N, BLK), jnp.float32),
                        pltpu.VMEM((R, NSLAB, BLK), jnp.float32),
                        pltpu.VMEM((R, NSLAB, BLK), jnp.int32)],
        compiler_params=pltpu.CompilerParams(
            dimension_semantics=("arbitrary",),
            vmem_limit_bytes=100 * 1024 * 1024),
    )


def _make_gather_max(residual):
    mesh = plsc.VectorSubcoreMesh(core_axis_name="c", subcore_axis_name="s")
    P = PTS_PER_WORKER
    scratch = [
        pltpu.VMEM((K, P), jnp.int32),
        pltpu.VMEM((P, 128), jnp.float32),
        pltpu.VMEM((P, 128), jnp.float32),
        pltpu.VMEM((P, 128), jnp.float32),
        pltpu.VMEM((P, 128), jnp.float32),
        pltpu.VMEM((P, OC), jnp.float32),
        pltpu.VMEM((P, OC), jnp.float32),
    ]
    if residual:
        scratch.append(pltpu.VMEM((P, OC), jnp.float32))
    scratch += [pltpu.SemaphoreType.DMA] * 4
    UNROLL = 4

    def body(*refs):
        if residual:
            (idxT_hbm, bf_hbm, a_hbm, xp_hbm, out_hbm, idx_v,
             buf0, buf1, buf2, buf3, acc, a_v, xp_v,
             sem0, sem1, sem2, sem3) = refs
        else:
            (idxT_hbm, bf_hbm, a_hbm, out_hbm, idx_v,
             buf0, buf1, buf2, buf3, acc, a_v,
             sem0, sem1, sem2, sem3) = refs
            xp_hbm = xp_v = None
        wid = lax.axis_index("s") * 2 + lax.axis_index("c")
        base = wid * P
        pltpu.sync_copy(idxT_hbm.at[:, pl.ds(base, P)], idx_v)
        pltpu.sync_copy(a_hbm.at[pl.ds(base, P)], a_v)
        if residual:
            pltpu.sync_copy(xp_hbm.at[pl.ds(base, P)], xp_v)

        bufs = (buf0, buf1, buf2, buf3)
        sems = (sem0, sem1, sem2, sem3)
        copies = [None] * 4
        for t in range(4):
            copies[t] = pltpu.async_copy(bf_hbm.at[idx_v.at[t]], bufs[t],
                                         sems[t])
        for t in range(0, K, 2):
            copies[t % 4].wait()
            copies[(t + 1) % 4].wait()
            ba, bb = bufs[t % 4], bufs[(t + 1) % 4]

            def pair_body(q, _, ba=ba, bb=bb, first=(t == 0)):
                for u in range(UNROLL):
                    p = q * UNROLL + u
                    for gch in range(OC // 16):
                        sl = pl.ds(gch * 16, 16)
                        v = jnp.maximum(ba[p, sl], bb[p, sl])
                        if not first:
                            v = jnp.maximum(acc[p, sl], v)
                        acc[p, sl] = v
                return 0
            lax.fori_loop(0, P // UNROLL, pair_body, 0)
            for tn in (t + 4, t + 5):
                if tn < K:
                    copies[tn % 4] = pltpu.async_copy(
                        bf_hbm.at[idx_v.at[tn]], bufs[tn % 4], sems[tn % 4])

        def epi_body(p, _):
            for gch in range(OC // 16):
                sl = pl.ds(gch * 16, 16)
                v = jnp.maximum(a_v[p, sl] + acc[p, sl], jnp.float32(0.0))
                if residual:
                    v = v - xp_v[p, sl]
                acc[p, sl] = v
            return 0
        lax.fori_loop(0, P, epi_body, 0)
        pltpu.sync_copy(acc, out_hbm.at[pl.ds(base, P)])

    return functools.partial(
        pl.kernel, body, mesh=mesh,
        out_type=jax.ShapeDtypeStruct((N, OC), jnp.float32),
        scratch_types=scratch)


def _head_body(x1_ref, x2_ref, x3_ref, A_ref, B_ref, C_ref, bl1_ref,
               Wm1_ref, bm1_ref, Wm2_ref, bm2_ref, Wm3_ref, bm3_ref,
               out_ref):
    dot = functools.partial(jnp.dot, preferred_element_type=jnp.float32)
    h = (dot(x1_ref[...], A_ref[...]) + dot(x2_ref[...], B_ref[...]) +
         dot(x3_ref[...], C_ref[...]) + bl1_ref[...])
    h = jnp.maximum(h, 0.0)
    h = jnp.maximum(dot(h, Wm1_ref[...]) + bm1_ref[...], 0.0)
    h = jnp.maximum(dot(h, Wm2_ref[...]) + bm2_ref[...], 0.0)
    logits = dot(h, Wm3_ref[...]) + bm3_ref[...]
    m = jnp.max(logits, axis=1, keepdims=True)
    s = jnp.sum(jnp.exp(logits - m), axis=1, keepdims=True)
    out_ref[...] = logits - (m + jnp.log(s))


def _head_call():
    full = lambda shape: pl.BlockSpec(shape, lambda i: (0, 0))
    return pl.pallas_call(
        _head_body,
        grid=(GRID,),
        in_specs=[
            pl.BlockSpec((BLK, OC), lambda i: (i, 0)),
            pl.BlockSpec((BLK, OC), lambda i: (i, 0)),
            pl.BlockSpec((BLK, OC), lambda i: (i, 0)),
            full((OC, 1024)), full((OC, 1024)), full((OC, 1024)),
            full((1, 1024)),
            full((1024, 256)), full((1, 256)),
            full((256, 128)), full((1, 128)),
            full((128, 64)), full((1, 64)),
        ],
        out_specs=pl.BlockSpec((BLK, 64), lambda i: (i, 0)),
        out_shape=jax.ShapeDtypeStruct((N, 64), jnp.float32),
        compiler_params=pltpu.CompilerParams(
            dimension_semantics=("arbitrary",)),
    )


def _split_edge_weights(W, C, pad_to):
    Wa, Wb = W[:C], W[C:]
    Wc = Wa - Wb
    if pad_to > C:
        padw = ((0, pad_to - C), (0, 0))
        Wc = jnp.pad(Wc, padw)
        Wb = jnp.pad(Wb, padw)
    Wb = jnp.pad(Wb, ((0, 0), (0, 128 - Wb.shape[1])))
    return Wc, Wb


def kernel(x, W1, b1, Wd1, bd1, Wd2, bd2, Wd3, bd3, Wl1, bl1, Wm1, bm1,
           Wm2, bm2, Wm3, bm3):
    gmax_plain = _make_gather_max(False)()
    gmax_res = _make_gather_max(True)()

    x0 = jnp.pad(x, ((0, 0), (0, 5)))
    Wc0, Wb0 = _split_edge_weights(W1, 3, 8)
    idxT0, a0, bf0 = _knn_call(8)(x0, x0.T, Wc0, Wb0, b1[None, :])
    f = gmax_plain(idxT0, bf0, a0)

    feats = []
    cur = f
    for Wd, bd in ((Wd1, bd1), (Wd2, bd2), (Wd3, bd3)):
        Wc, Wb = _split_edge_weights(Wd, OC, OC)
        idxT, a, bf = _knn_call(OC)(cur, cur.T, Wc, Wb, bd[None, :])
        nxt = gmax_res(idxT, bf, a, cur)
        feats.append(nxt)
        cur = nxt

    x1, x2, x3 = feats
    A, B, C = Wl1[:OC], Wl1[OC:2 * OC], Wl1[2 * OC:]
    Wm3p = jnp.pad(Wm3, ((0, 0), (0, 64 - Wm3.shape[1])))
    bm3p = jnp.pad(bm3, (0, 64 - bm3.shape[0]),
                   constant_values=-1e30)
    out = _head_call()(x1, x2, x3, A, B, C, bl1[None, :],
                       Wm1, bm1[None, :], Wm2, bm2[None, :],
                       Wm3p, bm3p[None, :])
    return out[:, :bm3.shape[0]]
